# Initial kernel scaffold; baseline (speedup 1.0000x reference)
import jax
import jax.numpy as jnp
from jax.experimental import pallas as pl


def kernel(h, x, edges, edge_attr, params):
    raise NotImplementedError("write your pallas kernel here")


# speedup vs baseline: 3.8385x; 3.8385x over previous
import functools

import jax
import jax.numpy as jnp
from jax import lax
from jax.experimental import pallas as pl
from jax.experimental.pallas import tpu as pltpu
from jax.experimental.pallas import tpu_sc as plsc

N = 10000
E = 320000
HID = 64
W = 80
XW = 16
NC, NS = 2, 16
NW = NC * NS
EPW = E // NW
CH = 80
NCHUNK = EPW // CH
GRP = 5
NGRP = NCHUNK // GRP
_RPT = N // NS


def _silu(v):
    return v * jax.nn.sigmoid(v)


def _drain(hbm_ref, vmem_ref, sem):
    pltpu.make_async_copy(hbm_ref, vmem_ref, sem).wait()


def _sc_gather_body(rowi_hbm, coli_hbm, ta_hbm, tb_hbm, ga_hbm, gb_hbm,
                    idxr, idxc, *rest):
    buf_a = rest[0:GRP]
    buf_b = rest[GRP:2 * GRP]
    semg = rest[2 * GRP:3 * GRP]
    semw = rest[3 * GRP:4 * GRP]
    wid = lax.axis_index("s") * NC + lax.axis_index("c")
    ebase = wid * EPW
    cbase = wid * NCHUNK
    pltpu.sync_copy(rowi_hbm.at[pl.ds(cbase, NCHUNK)], idxr)
    pltpu.sync_copy(coli_hbm.at[pl.ds(cbase, NCHUNK)], idxc)

    @pl.loop(0, NGRP)
    def _grp(j):
        @pl.when(j > 0)
        def _():
            for s in range(GRP):
                _drain(ta_hbm.at[pl.ds(0, CH)], buf_a[s], semw[s])
                _drain(tb_hbm.at[pl.ds(0, CH)], buf_b[s], semw[s])
        descs = []
        for s in range(GRP):
            ci = j * GRP + s
            descs.append(pltpu.async_copy(ta_hbm.at[idxr.at[ci]], buf_a[s], semg[s]))
            descs.append(pltpu.async_copy(tb_hbm.at[idxc.at[ci]], buf_b[s], semg[s]))
        for s in range(GRP):
            ci = j * GRP + s
            base = ebase + ci * CH
            descs[2 * s].wait()
            descs[2 * s + 1].wait()
            pltpu.async_copy(buf_a[s], ga_hbm.at[pl.ds(base, CH)], semw[s])
            pltpu.async_copy(buf_b[s], gb_hbm.at[pl.ds(base, CH)], semw[s])

    for s in range(GRP):
        _drain(ta_hbm.at[pl.ds(0, CH)], buf_a[s], semw[s])
        _drain(tb_hbm.at[pl.ds(0, CH)], buf_b[s], semw[s])


@functools.cache
def _get_sc_gather():
    mesh = plsc.VectorSubcoreMesh(
        core_axis_name="c", subcore_axis_name="s",
        num_cores=NC, num_subcores=NS)
    return pl.kernel(
        _sc_gather_body,
        out_type=(
            jax.ShapeDtypeStruct((E, W), jnp.float32),
            jax.ShapeDtypeStruct((E, W), jnp.float32),
        ),
        mesh=mesh,
        compiler_params=pltpu.CompilerParams(use_tc_tiling_on_sc=False),
        scratch_types=(
            [pltpu.VMEM((NCHUNK, CH), jnp.int32)] * 2
            + [pltpu.VMEM((CH, W), jnp.float32)] * (2 * GRP)
            + [pltpu.SemaphoreType.DMA] * (2 * GRP)
        ),
    )


def _sc_gather(*args):
    return _get_sc_gather()(*args)


def _sc_scatter_body(rowi_hbm, o_hbm, z80_hbm, p80_hbm, acc80, idxr, *rest):
    buf = rest[0:GRP]
    seml = rest[GRP:2 * GRP]
    sema = rest[2 * GRP:3 * GRP]
    cid = lax.axis_index("c")
    sid = lax.axis_index("s")
    wid = sid * NC + cid
    ebase = wid * EPW
    cbase = wid * NCHUNK
    r0 = sid * _RPT
    pltpu.sync_copy(z80_hbm.at[pl.ds(r0, _RPT)], acc80.at[pl.ds(r0, _RPT)])
    pltpu.sync_copy(rowi_hbm.at[pl.ds(cbase, NCHUNK)], idxr)
    plsc.subcore_barrier()

    @pl.loop(0, NGRP)
    def _grp(j):
        @pl.when(j > 0)
        def _():
            for s in range(GRP):
                _drain(o_hbm.at[pl.ds(0, CH)], buf[s], sema[s])
        descs = []
        for s in range(GRP):
            base = ebase + (j * GRP + s) * CH
            descs.append(pltpu.async_copy(o_hbm.at[pl.ds(base, CH)], buf[s], seml[s]))
        for s in range(GRP):
            ci = j * GRP + s
            descs[s].wait()
            pltpu.async_copy(buf[s], acc80.at[idxr.at[ci]], sema[s], add=True)

    for s in range(GRP):
        _drain(o_hbm.at[pl.ds(0, CH)], buf[s], sema[s])
    plsc.subcore_barrier()
    pltpu.sync_copy(acc80.at[pl.ds(r0, _RPT)], p80_hbm.at[cid, pl.ds(r0, _RPT)])


@functools.cache
def _get_sc_scatter():
    mesh = plsc.VectorSubcoreMesh(
        core_axis_name="c", subcore_axis_name="s",
        num_cores=NC, num_subcores=NS)
    return pl.kernel(
        _sc_scatter_body,
        out_type=jax.ShapeDtypeStruct((NC, N, W), jnp.float32),
        mesh=mesh,
        compiler_params=pltpu.CompilerParams(use_tc_tiling_on_sc=False),
        scratch_types=(
            [pltpu.VMEM_SHARED((N, W), jnp.float32),
             pltpu.VMEM((NCHUNK, CH), jnp.int32)]
            + [pltpu.VMEM((CH, W), jnp.float32)] * GRP
            + [pltpu.SemaphoreType.DMA] * (2 * GRP)
        ),
    )


def _sc_scatter(*args):
    return _get_sc_scatter()(*args)


_NB = 2000
_NGRID = N // _NB
_EB = 6400
_EGRID = E // _EB


def _full(shape):
    return pl.BlockSpec(shape, lambda i: tuple(0 for _ in shape))


def _dot(a, b):
    return jnp.dot(a, b, preferred_element_type=jnp.float32)


def _embed_body(h_ref, w_ref, b_ref, o_ref):
    o_ref[...] = _dot(h_ref[...], w_ref[...]) + b_ref[...]


def _mm_call(in_nf, out_nf):
    return pl.pallas_call(
        _embed_body,
        grid=(_NGRID,),
        in_specs=[
            pl.BlockSpec((_NB, in_nf), lambda i: (i, 0)),
            _full((in_nf, out_nf)),
            _full((1, out_nf)),
        ],
        out_specs=pl.BlockSpec((_NB, out_nf), lambda i: (i, 0)),
        out_shape=jax.ShapeDtypeStruct((N, out_nf), jnp.float32),
    )


_embed_in = _mm_call(128, HID)
_embed_out = _mm_call(HID, 128)


def _prep_body(h_ref, x_ref, w0a_ref, w0b_ref, b0_ref, ta_ref, tb_ref):
    hv = h_ref[...]
    xv = x_ref[...]
    ta_ref[...] = jnp.concatenate(
        [_dot(hv, w0a_ref[...]) + b0_ref[...], xv], axis=1)
    tb_ref[...] = jnp.concatenate([_dot(hv, w0b_ref[...]), -xv], axis=1)


_prep = pl.pallas_call(
    _prep_body,
    grid=(_NGRID,),
    in_specs=[
        pl.BlockSpec((_NB, HID), lambda i: (i, 0)),
        pl.BlockSpec((_NB, XW), lambda i: (i, 0)),
        _full((HID, HID)),
        _full((HID, HID)),
        _full((1, HID)),
    ],
    out_specs=[
        pl.BlockSpec((_NB, W), lambda i: (i, 0)),
        pl.BlockSpec((_NB, W), lambda i: (i, 0)),
    ],
    out_shape=[
        jax.ShapeDtypeStruct((N, W), jnp.float32),
        jax.ShapeDtypeStruct((N, W), jnp.float32),
    ],
)


def _edge_body(ga_ref, gb_ref, ea_ref,
               w0r_ref, w0e_ref, w1_ref, b1_ref, wc0_ref, bc0_ref, wc1_ref,
               o_ref):
    sv = ga_ref[...] + gb_ref[...]
    lane = lax.broadcasted_iota(jnp.int32, (_EB, W), 1)
    mx = (lane >= HID).astype(jnp.float32)
    dm = sv * mx
    radial = jnp.sum(dm * dm, axis=1, keepdims=True)
    z1 = sv + radial * w0r_ref[...] + _dot(ea_ref[...], w0e_ref[...])
    e1 = _silu(z1)
    ef = _silu(_dot(e1, w1_ref[...]) + b1_ref[...])
    t = _silu(_dot(ef, wc0_ref[...]) + bc0_ref[...])
    cm = jnp.sum(t * wc1_ref[...], axis=1, keepdims=True)
    o_ref[...] = ef + dm * (cm / (jnp.sqrt(radial + 1e-8) + 1.0))


_edge = pl.pallas_call(
    _edge_body,
    grid=(_EGRID,),
    in_specs=[
        pl.BlockSpec((_EB, W), lambda i: (i, 0)),
        pl.BlockSpec((_EB, W), lambda i: (i, 0)),
        pl.BlockSpec((_EB, 8), lambda i: (i, 0)),
        _full((1, W)),
        _full((8, W)),
        _full((W, W)),
        _full((1, W)),
        _full((W, HID)),
        _full((1, HID)),
        _full((1, HID)),
    ],
    out_specs=pl.BlockSpec((_EB, W), lambda i: (i, 0)),
    out_shape=jax.ShapeDtypeStruct((E, W), jnp.float32),
)


def _node_body(h_ref, p80_ref, x_ref,
               wn0a_ref, wn0b_ref, bn0_ref, wn1_ref, bn1_ref,
               ho_ref, xo_ref):
    hv = h_ref[...]
    s80 = p80_ref[0] + p80_ref[1]
    nm = _silu(_dot(hv, wn0a_ref[...]) + _dot(s80, wn0b_ref[...]) + bn0_ref[...])
    nm = _dot(nm, wn1_ref[...]) + bn1_ref[...]
    ho_ref[...] = hv + nm
    xo_ref[...] = x_ref[...] + s80[:, HID:W]


_node = pl.pallas_call(
    _node_body,
    grid=(_NGRID,),
    in_specs=[
        pl.BlockSpec((_NB, HID), lambda i: (i, 0)),
        pl.BlockSpec((NC, _NB, W), lambda i: (0, i, 0)),
        pl.BlockSpec((_NB, XW), lambda i: (i, 0)),
        _full((HID, HID)),
        _full((W, HID)),
        _full((1, HID)),
        _full((HID, HID)),
        _full((1, HID)),
    ],
    out_specs=[
        pl.BlockSpec((_NB, HID), lambda i: (i, 0)),
        pl.BlockSpec((_NB, XW), lambda i: (i, 0)),
    ],
    out_shape=[
        jax.ShapeDtypeStruct((N, HID), jnp.float32),
        jax.ShapeDtypeStruct((N, XW), jnp.float32),
    ],
)


def kernel(h, x, edges, edge_attr, params):
    row = edges[0]
    col = edges[1]
    rowi = row.reshape(E // CH, CH)
    coli = col.reshape(E // CH, CH)
    xp = jnp.pad(x, ((0, 0), (0, XW - 3)))
    ea8 = jnp.pad(edge_attr, ((0, 0), (0, 4)))
    z80 = jnp.zeros((N, W), jnp.float32)

    def b2d(b):
        return b.reshape(1, -1)

    hcur = _embed_in(h, params["emb_in"]["w"], b2d(params["emb_in"]["b"]))
    xcur = xp
    for lp in params["layers"]:
        w0 = lp["edge_mlp0"]["w"]
        w0a, w0b = w0[0:HID], w0[HID:2 * HID]
        w0r80 = jnp.pad(w0[2 * HID:2 * HID + 1], ((0, 0), (0, XW)))
        w0e80 = jnp.pad(w0[2 * HID + 1:], ((0, 4), (0, XW)))
        w1e = jnp.pad(lp["edge_mlp1"]["w"], ((0, XW), (0, XW)))
        b1e = jnp.pad(b2d(lp["edge_mlp1"]["b"]), ((0, 0), (0, XW)))
        wc0e = jnp.pad(lp["coord_mlp0"]["w"], ((0, XW), (0, 0)))
        ta, tb = _prep(hcur, xcur, w0a, w0b, b2d(lp["edge_mlp0"]["b"]))
        ga, gb = _sc_gather(rowi, coli, ta, tb)
        o = _edge(ga, gb, ea8,
                  w0r80, w0e80, w1e, b1e,
                  wc0e, b2d(lp["coord_mlp0"]["b"]),
                  lp["coord_mlp1"]["w"].reshape(1, HID))
        p80 = _sc_scatter(rowi, o, z80)
        nw0 = lp["node_mlp0"]["w"]
        wn0be = jnp.pad(nw0[HID:], ((0, XW), (0, 0)))
        hcur, xcur = _node(hcur, p80, xcur,
                           nw0[:HID], wn0be, b2d(lp["node_mlp0"]["b"]),
                           lp["node_mlp1"]["w"], b2d(lp["node_mlp1"]["b"]))
    hout = _embed_out(hcur, params["emb_out"]["w"], b2d(params["emb_out"]["b"]))
    return (hout, xcur[:, :3])


# speedup vs baseline: 6.0086x; 1.5653x over previous
import functools

import jax
import jax.numpy as jnp
from jax import lax
from jax.experimental import pallas as pl
from jax.experimental.pallas import tpu as pltpu
from jax.experimental.pallas import tpu_sc as plsc

N = 10000
E = 320000
HID = 64
W = 128
XW = 16
NC, NS = 2, 16
NW = NC * NS
EPW = E // NW

CHG = 80
NCHG = EPW // CHG
GRPG = 5
NGRPG = NCHG // GRPG

CHS = 128
NFULL = EPW // CHS
TAIL = EPW - NFULL * CHS
SPW = NFULL + 1
GRPS = 2
NGRPS = NFULL // GRPS
NACC = N + 8


def _silu(v):
    return v * jax.nn.sigmoid(v)


def _drain(hbm_ref, vmem_ref, sem):
    pltpu.make_async_copy(hbm_ref, vmem_ref, sem).wait()


def _sc_gather_body(row_hbm, col_hbm, ta_hbm, tb_hbm, ga_hbm, gb_hbm,
                    idxr, idxc, *rest):
    buf_a = rest[0:GRPG]
    buf_b = rest[GRPG:2 * GRPG]
    semg = rest[2 * GRPG:3 * GRPG]
    semw = rest[3 * GRPG:4 * GRPG]
    wid = lax.axis_index("s") * NC + lax.axis_index("c")
    ebase = wid * EPW
    pltpu.sync_copy(row_hbm.at[pl.ds(ebase, EPW)], idxr)
    pltpu.sync_copy(col_hbm.at[pl.ds(ebase, EPW)], idxc)

    @pl.loop(0, NGRPG)
    def _grp(j):
        @pl.when(j > 0)
        def _():
            for s in range(GRPG):
                _drain(ta_hbm.at[pl.ds(0, CHG)], buf_a[s], semw[s])
                _drain(tb_hbm.at[pl.ds(0, CHG)], buf_b[s], semw[s])
        descs = []
        for s in range(GRPG):
            off = (j * GRPG + s) * CHG
            descs.append(pltpu.async_copy(
                ta_hbm.at[idxr.at[pl.ds(off, CHG)]], buf_a[s], semg[s]))
            descs.append(pltpu.async_copy(
                tb_hbm.at[idxc.at[pl.ds(off, CHG)]], buf_b[s], semg[s]))
        for s in range(GRPG):
            base = ebase + (j * GRPG + s) * CHG
            descs[2 * s].wait()
            descs[2 * s + 1].wait()
            pltpu.async_copy(buf_a[s], ga_hbm.at[pl.ds(base, CHG)], semw[s])
            pltpu.async_copy(buf_b[s], gb_hbm.at[pl.ds(base, CHG)], semw[s])

    for s in range(GRPG):
        _drain(ta_hbm.at[pl.ds(0, CHG)], buf_a[s], semw[s])
        _drain(tb_hbm.at[pl.ds(0, CHG)], buf_b[s], semw[s])


@functools.cache
def _get_sc_gather():
    mesh = plsc.VectorSubcoreMesh(
        core_axis_name="c", subcore_axis_name="s",
        num_cores=NC, num_subcores=NS)
    return pl.kernel(
        _sc_gather_body,
        out_type=(
            jax.ShapeDtypeStruct((E, W), jnp.float32),
            jax.ShapeDtypeStruct((E, W), jnp.float32),
        ),
        mesh=mesh,
        scratch_types=(
            [pltpu.VMEM((EPW,), jnp.int32)] * 2
            + [pltpu.VMEM((CHG, W), jnp.float32)] * (2 * GRPG)
            + [pltpu.SemaphoreType.DMA] * (2 * GRPG)
        ),
    )


def _sc_gather(*args):
    return _get_sc_gather()(*args)


def _sc_scatter_body(rowp_hbm, o_hbm, zacc_hbm, p_hbm, acc, idxr, *rest):
    buf = rest[0:GRPS]
    seml = rest[GRPS:2 * GRPS]
    sema = rest[2 * GRPS:3 * GRPS]
    cid = lax.axis_index("c")
    sid = lax.axis_index("s")
    wid = sid * NC + cid
    ebase = wid * EPW

    @pl.when(sid == 0)
    def _():
        pltpu.sync_copy(zacc_hbm, acc)
    pltpu.sync_copy(rowp_hbm.at[wid], idxr)
    plsc.subcore_barrier()

    @pl.loop(0, NGRPS)
    def _grp(j):
        @pl.when(j > 0)
        def _():
            for s in range(GRPS):
                _drain(o_hbm.at[pl.ds(0, CHS)], buf[s], sema[s])
        descs = []
        for s in range(GRPS):
            base = ebase + (j * GRPS + s) * CHS
            descs.append(pltpu.async_copy(o_hbm.at[pl.ds(base, CHS)], buf[s], seml[s]))
        for s in range(GRPS):
            ci = j * GRPS + s
            descs[s].wait()
            pltpu.async_copy(buf[s], acc.at[idxr.at[ci]], sema[s], add=True)

    for s in range(GRPS):
        _drain(o_hbm.at[pl.ds(0, CHS)], buf[s], sema[s])
    pltpu.sync_copy(o_hbm.at[pl.ds(ebase + NFULL * CHS, TAIL)],
                    buf[0].at[pl.ds(0, TAIL)])
    pltpu.sync_copy(buf[0], acc.at[idxr.at[NFULL]], add=True)
    plsc.subcore_barrier()

    @pl.when(sid < 2)
    def _():
        r0 = sid * 632
        pltpu.sync_copy(acc.at[pl.ds(r0, 632)], p_hbm.at[cid, pl.ds(r0, 632)])

    @pl.when(sid >= 2)
    def _():
        r0 = 1264 + (sid - 2) * 624
        pltpu.sync_copy(acc.at[pl.ds(r0, 624)], p_hbm.at[cid, pl.ds(r0, 624)])


@functools.cache
def _get_sc_scatter():
    mesh = plsc.VectorSubcoreMesh(
        core_axis_name="c", subcore_axis_name="s",
        num_cores=NC, num_subcores=NS)
    return pl.kernel(
        _sc_scatter_body,
        out_type=jax.ShapeDtypeStruct((NC, N, W), jnp.float32),
        mesh=mesh,
        scratch_types=(
            [pltpu.VMEM_SHARED((NACC, W), jnp.float32),
             pltpu.VMEM((SPW, CHS), jnp.int32)]
            + [pltpu.VMEM((CHS, W), jnp.float32)] * GRPS
            + [pltpu.SemaphoreType.DMA] * (2 * GRPS)
        ),
    )


def _sc_scatter(*args):
    return _get_sc_scatter()(*args)


_NB = 2000
_NGRID = N // _NB
_EB = 6400
_EGRID = E // _EB


def _full(shape):
    return pl.BlockSpec(shape, lambda i: tuple(0 for _ in shape))


def _dot(a, b):
    return jnp.dot(a, b, preferred_element_type=jnp.float32)


def _embed_body(h_ref, w_ref, b_ref, o_ref):
    o_ref[...] = _dot(h_ref[...], w_ref[...]) + b_ref[...]


def _mm_call(in_nf, out_nf):
    return pl.pallas_call(
        _embed_body,
        grid=(_NGRID,),
        in_specs=[
            pl.BlockSpec((_NB, in_nf), lambda i: (i, 0)),
            _full((in_nf, out_nf)),
            _full((1, out_nf)),
        ],
        out_specs=pl.BlockSpec((_NB, out_nf), lambda i: (i, 0)),
        out_shape=jax.ShapeDtypeStruct((N, out_nf), jnp.float32),
    )


_embed_in = _mm_call(128, HID)
_embed_out = _mm_call(HID, 128)


def _prep_body(h_ref, x_ref, w0a_ref, w0b_ref, b0_ref, ta_ref, tb_ref):
    hv = h_ref[...]
    xv = x_ref[...]
    z = jnp.zeros((_NB, W - HID - XW), jnp.float32)
    ta_ref[...] = jnp.concatenate(
        [_dot(hv, w0a_ref[...]) + b0_ref[...], xv, z], axis=1)
    tb_ref[...] = jnp.concatenate([_dot(hv, w0b_ref[...]), -xv, z], axis=1)


_prep = pl.pallas_call(
    _prep_body,
    grid=(_NGRID,),
    in_specs=[
        pl.BlockSpec((_NB, HID), lambda i: (i, 0)),
        pl.BlockSpec((_NB, XW), lambda i: (i, 0)),
        _full((HID, HID)),
        _full((HID, HID)),
        _full((1, HID)),
    ],
    out_specs=[
        pl.BlockSpec((_NB, W), lambda i: (i, 0)),
        pl.BlockSpec((_NB, W), lambda i: (i, 0)),
    ],
    out_shape=[
        jax.ShapeDtypeStruct((N, W), jnp.float32),
        jax.ShapeDtypeStruct((N, W), jnp.float32),
    ],
)


def _edge_body(ga_ref, gb_ref, ea_ref,
               w0r_ref, w0e_ref, w1_ref, b1_ref, wc0_ref, bc0_ref, wc1_ref,
               o_ref):
    sv = ga_ref[...] + gb_ref[...]
    lane = lax.broadcasted_iota(jnp.int32, (_EB, W), 1)
    mx = ((lane >= HID) & (lane < HID + XW)).astype(jnp.float32)
    dm = sv * mx
    radial = jnp.sum(dm * dm, axis=1, keepdims=True)
    z1 = sv + radial * w0r_ref[...] + _dot(ea_ref[...], w0e_ref[...])
    e1 = _silu(z1)
    ef = _silu(_dot(e1, w1_ref[...]) + b1_ref[...])
    t = _silu(_dot(ef, wc0_ref[...]) + bc0_ref[...])
    cm = jnp.sum(t * wc1_ref[...], axis=1, keepdims=True)
    o_ref[...] = ef + dm * (cm / (jnp.sqrt(radial + 1e-8) + 1.0))


_edge = pl.pallas_call(
    _edge_body,
    grid=(_EGRID,),
    in_specs=[
        pl.BlockSpec((_EB, W), lambda i: (i, 0)),
        pl.BlockSpec((_EB, W), lambda i: (i, 0)),
        pl.BlockSpec((_EB, 4), lambda i: (i, 0)),
        _full((1, W)),
        _full((4, W)),
        _full((W, W)),
        _full((1, W)),
        _full((W, HID)),
        _full((1, HID)),
        _full((1, HID)),
    ],
    out_specs=pl.BlockSpec((_EB, W), lambda i: (i, 0)),
    out_shape=jax.ShapeDtypeStruct((E, W), jnp.float32),
)


def _node_body(h_ref, p_ref, x_ref,
               wn0a_ref, wn0b_ref, bn0_ref, wn1_ref, bn1_ref,
               ho_ref, xo_ref):
    hv = h_ref[...]
    sp = p_ref[0] + p_ref[1]
    nm = _silu(_dot(hv, wn0a_ref[...]) + _dot(sp, wn0b_ref[...]) + bn0_ref[...])
    nm = _dot(nm, wn1_ref[...]) + bn1_ref[...]
    ho_ref[...] = hv + nm
    xo_ref[...] = x_ref[...] + sp[:, HID:HID + XW]


_node = pl.pallas_call(
    _node_body,
    grid=(_NGRID,),
    in_specs=[
        pl.BlockSpec((_NB, HID), lambda i: (i, 0)),
        pl.BlockSpec((NC, _NB, W), lambda i: (0, i, 0)),
        pl.BlockSpec((_NB, XW), lambda i: (i, 0)),
        _full((HID, HID)),
        _full((W, HID)),
        _full((1, HID)),
        _full((HID, HID)),
        _full((1, HID)),
    ],
    out_specs=[
        pl.BlockSpec((_NB, HID), lambda i: (i, 0)),
        pl.BlockSpec((_NB, XW), lambda i: (i, 0)),
    ],
    out_shape=[
        jax.ShapeDtypeStruct((N, HID), jnp.float32),
        jax.ShapeDtypeStruct((N, XW), jnp.float32),
    ],
)


def kernel(h, x, edges, edge_attr, params):
    row = edges[0]
    col = edges[1]
    rowp = jnp.pad(row.reshape(NW, EPW), ((0, 0), (0, SPW * CHS - EPW)),
                   constant_values=N).reshape(NW, SPW, CHS)
    xp = jnp.pad(x, ((0, 0), (0, XW - 3)))
    zacc = jnp.zeros((NACC, W), jnp.float32)

    def b2d(b):
        return b.reshape(1, -1)

    def padw(m):
        return jnp.pad(m, ((0, 0), (0, W - m.shape[1])))

    hcur = _embed_in(h, params["emb_in"]["w"], b2d(params["emb_in"]["b"]))
    xcur = xp
    for lp in params["layers"]:
        w0 = lp["edge_mlp0"]["w"]
        w0a, w0b = w0[0:HID], w0[HID:2 * HID]
        w0r = padw(w0[2 * HID:2 * HID + 1])
        w0e = padw(w0[2 * HID + 1:])
        w1e = jnp.pad(lp["edge_mlp1"]["w"],
                      ((0, W - HID), (0, W - HID)))
        b1e = padw(b2d(lp["edge_mlp1"]["b"]))
        wc0e = jnp.pad(lp["coord_mlp0"]["w"], ((0, W - HID), (0, 0)))
        ta, tb = _prep(hcur, xcur, w0a, w0b, b2d(lp["edge_mlp0"]["b"]))
        ga, gb = _sc_gather(row, col, ta, tb)
        o = _edge(ga, gb, edge_attr,
                  w0r, w0e, w1e, b1e,
                  wc0e, b2d(lp["coord_mlp0"]["b"]),
                  lp["coord_mlp1"]["w"].reshape(1, HID))
        p = _sc_scatter(rowp, o, zacc)
        nw0 = lp["node_mlp0"]["w"]
        wn0be = jnp.pad(nw0[HID:], ((0, W - HID), (0, 0)))
        hcur, xcur = _node(hcur, p, xcur,
                           nw0[:HID], wn0be, b2d(lp["node_mlp0"]["b"]),
                           lp["node_mlp1"]["w"], b2d(lp["node_mlp1"]["b"]))
    hout = _embed_out(hcur, params["emb_out"]["w"], b2d(params["emb_out"]["b"]))
    return (hout, xcur[:, :3])


# speedup vs baseline: 6.4204x; 1.0685x over previous
import functools

import jax
import jax.numpy as jnp
from jax import lax
from jax.experimental import pallas as pl
from jax.experimental.pallas import tpu as pltpu
from jax.experimental.pallas import tpu_sc as plsc

N = 10000
E = 320000
HID = 64
W = 128
XW = 16
NC, NS = 2, 16
NW = NC * NS
EPW = E // NW

EA = 192000
EB = E - EA
EPWA = EA // NW
EPWB = EB // NW

CHG = 80
GRPG = 5

CHS = 128
GRPS = 2
NACC = N + 8


def _silu(v):
    return v * jax.nn.sigmoid(v)


def _drain(hbm_ref, vmem_ref, sem):
    pltpu.make_async_copy(hbm_ref, vmem_ref, sem).wait()


@functools.cache
def _get_sc_gather(epw):
    nchg = epw // CHG
    ngrpg = nchg // GRPG
    ne = epw * NW

    def body(row_hbm, col_hbm, ta_hbm, tb_hbm, ga_hbm, gb_hbm,
             idxr, idxc, *rest):
        buf_a = rest[0:GRPG]
        buf_b = rest[GRPG:2 * GRPG]
        semg = rest[2 * GRPG:3 * GRPG]
        semw = rest[3 * GRPG:4 * GRPG]
        wid = lax.axis_index("s") * NC + lax.axis_index("c")
        ebase = wid * epw
        pltpu.sync_copy(row_hbm.at[pl.ds(ebase, epw)], idxr)
        pltpu.sync_copy(col_hbm.at[pl.ds(ebase, epw)], idxc)

        @pl.loop(0, ngrpg)
        def _grp(j):
            @pl.when(j > 0)
            def _():
                for s in range(GRPG):
                    _drain(ta_hbm.at[pl.ds(0, CHG)], buf_a[s], semw[s])
                    _drain(tb_hbm.at[pl.ds(0, CHG)], buf_b[s], semw[s])
            descs = []
            for s in range(GRPG):
                off = (j * GRPG + s) * CHG
                descs.append(pltpu.async_copy(
                    ta_hbm.at[idxr.at[pl.ds(off, CHG)]], buf_a[s], semg[s]))
                descs.append(pltpu.async_copy(
                    tb_hbm.at[idxc.at[pl.ds(off, CHG)]], buf_b[s], semg[s]))
            for s in range(GRPG):
                base = ebase + (j * GRPG + s) * CHG
                descs[2 * s].wait()
                descs[2 * s + 1].wait()
                pltpu.async_copy(buf_a[s], ga_hbm.at[pl.ds(base, CHG)], semw[s])
                pltpu.async_copy(buf_b[s], gb_hbm.at[pl.ds(base, CHG)], semw[s])

        for s in range(GRPG):
            _drain(ta_hbm.at[pl.ds(0, CHG)], buf_a[s], semw[s])
            _drain(tb_hbm.at[pl.ds(0, CHG)], buf_b[s], semw[s])

    mesh = plsc.VectorSubcoreMesh(
        core_axis_name="c", subcore_axis_name="s",
        num_cores=NC, num_subcores=NS)
    return pl.kernel(
        body,
        out_type=(
            jax.ShapeDtypeStruct((ne, W), jnp.float32),
            jax.ShapeDtypeStruct((ne, W), jnp.float32),
        ),
        mesh=mesh,
        scratch_types=(
            [pltpu.VMEM((epw,), jnp.int32)] * 2
            + [pltpu.VMEM((CHG, W), jnp.float32)] * (2 * GRPG)
            + [pltpu.SemaphoreType.DMA] * (2 * GRPG)
        ),
    )


def _sc_gather(epw, *args):
    return _get_sc_gather(epw)(*args)


@functools.cache
def _get_sc_scatter(epw):
    nfull = epw // CHS
    tail = epw - nfull * CHS
    spw = nfull + (1 if tail else 0)
    ngrps = nfull // GRPS
    rem = nfull - ngrps * GRPS

    def body(rowp_hbm, o_hbm, zacc_hbm, p_hbm, acc, idxr, *rest):
        buf = rest[0:GRPS]
        seml = rest[GRPS:2 * GRPS]
        sema = rest[2 * GRPS:3 * GRPS]
        cid = lax.axis_index("c")
        sid = lax.axis_index("s")
        wid = sid * NC + cid
        ebase = wid * epw

        @pl.when(sid == 0)
        def _():
            pltpu.sync_copy(zacc_hbm, acc)
        pltpu.sync_copy(rowp_hbm.at[wid], idxr)
        plsc.subcore_barrier()

        @pl.loop(0, ngrps)
        def _grp(j):
            @pl.when(j > 0)
            def _():
                for s in range(GRPS):
                    _drain(o_hbm.at[pl.ds(0, CHS)], buf[s], sema[s])
            descs = []
            for s in range(GRPS):
                base = ebase + (j * GRPS + s) * CHS
                descs.append(
                    pltpu.async_copy(o_hbm.at[pl.ds(base, CHS)], buf[s], seml[s]))
            for s in range(GRPS):
                ci = j * GRPS + s
                descs[s].wait()
                pltpu.async_copy(buf[s], acc.at[idxr.at[ci]], sema[s], add=True)

        for s in range(GRPS):
            _drain(o_hbm.at[pl.ds(0, CHS)], buf[s], sema[s])
        for k in range(rem):
            ci = ngrps * GRPS + k
            pltpu.sync_copy(o_hbm.at[pl.ds(ebase + ci * CHS, CHS)], buf[0])
            pltpu.sync_copy(buf[0], acc.at[idxr.at[ci]], add=True)
        if tail:
            pltpu.sync_copy(o_hbm.at[pl.ds(ebase + nfull * CHS, tail)],
                            buf[0].at[pl.ds(0, tail)])
            pltpu.sync_copy(buf[0], acc.at[idxr.at[nfull]], add=True)
        plsc.subcore_barrier()

        @pl.when(sid < 2)
        def _():
            r0 = sid * 632
            pltpu.sync_copy(acc.at[pl.ds(r0, 632)], p_hbm.at[cid, pl.ds(r0, 632)])

        @pl.when(sid >= 2)
        def _():
            r0 = 1264 + (sid - 2) * 624
            pltpu.sync_copy(acc.at[pl.ds(r0, 624)], p_hbm.at[cid, pl.ds(r0, 624)])

    mesh = plsc.VectorSubcoreMesh(
        core_axis_name="c", subcore_axis_name="s",
        num_cores=NC, num_subcores=NS)
    return pl.kernel(
        body,
        out_type=jax.ShapeDtypeStruct((NC, N, W), jnp.float32),
        mesh=mesh,
        scratch_types=(
            [pltpu.VMEM_SHARED((NACC, W), jnp.float32),
             pltpu.VMEM((spw, CHS), jnp.int32)]
            + [pltpu.VMEM((CHS, W), jnp.float32)] * GRPS
            + [pltpu.SemaphoreType.DMA] * (2 * GRPS)
        ),
    )


def _sc_scatter(epw, *args):
    return _get_sc_scatter(epw)(*args)


_NB = 2000
_NGRID = N // _NB
_EB = 6400
_EGRID = E // _EB


def _full(shape):
    return pl.BlockSpec(shape, lambda i: tuple(0 for _ in shape))


def _dot(a, b):
    return jnp.dot(a, b, preferred_element_type=jnp.float32)


def _embed_body(h_ref, w_ref, b_ref, o_ref):
    o_ref[...] = _dot(h_ref[...], w_ref[...]) + b_ref[...]


def _mm_call(in_nf, out_nf):
    return pl.pallas_call(
        _embed_body,
        grid=(_NGRID,),
        in_specs=[
            pl.BlockSpec((_NB, in_nf), lambda i: (i, 0)),
            _full((in_nf, out_nf)),
            _full((1, out_nf)),
        ],
        out_specs=pl.BlockSpec((_NB, out_nf), lambda i: (i, 0)),
        out_shape=jax.ShapeDtypeStruct((N, out_nf), jnp.float32),
    )


_embed_in = _mm_call(128, HID)
_embed_out = _mm_call(HID, 128)


def _prep_body(h_ref, x_ref, w0a_ref, w0b_ref, b0_ref, ta_ref, tb_ref):
    hv = h_ref[...]
    xv = x_ref[...]
    z = jnp.zeros((_NB, W - HID - XW), jnp.float32)
    ta_ref[...] = jnp.concatenate(
        [_dot(hv, w0a_ref[...]) + b0_ref[...], xv, z], axis=1)
    tb_ref[...] = jnp.concatenate([_dot(hv, w0b_ref[...]), -xv, z], axis=1)


_prep = pl.pallas_call(
    _prep_body,
    grid=(_NGRID,),
    in_specs=[
        pl.BlockSpec((_NB, HID), lambda i: (i, 0)),
        pl.BlockSpec((_NB, XW), lambda i: (i, 0)),
        _full((HID, HID)),
        _full((HID, HID)),
        _full((1, HID)),
    ],
    out_specs=[
        pl.BlockSpec((_NB, W), lambda i: (i, 0)),
        pl.BlockSpec((_NB, W), lambda i: (i, 0)),
    ],
    out_shape=[
        jax.ShapeDtypeStruct((N, W), jnp.float32),
        jax.ShapeDtypeStruct((N, W), jnp.float32),
    ],
)


def _edge_body(ga_ref, gb_ref, ea_ref,
               w0r_ref, w0e_ref, w1_ref, b1_ref, wc0_ref, bc0_ref, wc1_ref,
               o_ref):
    sv = ga_ref[...] + gb_ref[...]
    lane = lax.broadcasted_iota(jnp.int32, (_EB, W), 1)
    mx = ((lane >= HID) & (lane < HID + XW)).astype(jnp.float32)
    dm = sv * mx
    radial = jnp.sum(dm * dm, axis=1, keepdims=True)
    z1 = sv + radial * w0r_ref[...] + _dot(ea_ref[...], w0e_ref[...])
    e1 = _silu(z1)
    ef = _silu(_dot(e1, w1_ref[...]) + b1_ref[...])
    t = _silu(_dot(ef, wc0_ref[...]) + bc0_ref[...])
    cm = jnp.sum(t * wc1_ref[...], axis=1, keepdims=True)
    o_ref[...] = ef + dm * (cm / (jnp.sqrt(radial + 1e-8) + 1.0))


@functools.cache
def _get_edge(ne):
    return pl.pallas_call(
        _edge_body,
        grid=(ne // _EB,),
        in_specs=[
            pl.BlockSpec((_EB, W), lambda i: (i, 0)),
            pl.BlockSpec((_EB, W), lambda i: (i, 0)),
            pl.BlockSpec((_EB, 4), lambda i: (i, 0)),
            _full((1, W)),
            _full((4, W)),
            _full((W, W)),
            _full((1, W)),
            _full((W, HID)),
            _full((1, HID)),
            _full((1, HID)),
        ],
        out_specs=pl.BlockSpec((_EB, W), lambda i: (i, 0)),
        out_shape=jax.ShapeDtypeStruct((ne, W), jnp.float32),
    )


def _node_body(h_ref, p_ref, q_ref, x_ref,
               wn0a_ref, wn0b_ref, bn0_ref, wn1_ref, bn1_ref,
               ho_ref, xo_ref):
    hv = h_ref[...]
    sp = (p_ref[0] + p_ref[1]) + (q_ref[0] + q_ref[1])
    nm = _silu(_dot(hv, wn0a_ref[...]) + _dot(sp, wn0b_ref[...]) + bn0_ref[...])
    nm = _dot(nm, wn1_ref[...]) + bn1_ref[...]
    ho_ref[...] = hv + nm
    xo_ref[...] = x_ref[...] + sp[:, HID:HID + XW]


_node = pl.pallas_call(
    _node_body,
    grid=(_NGRID,),
    in_specs=[
        pl.BlockSpec((_NB, HID), lambda i: (i, 0)),
        pl.BlockSpec((NC, _NB, W), lambda i: (0, i, 0)),
        pl.BlockSpec((NC, _NB, W), lambda i: (0, i, 0)),
        pl.BlockSpec((_NB, XW), lambda i: (i, 0)),
        _full((HID, HID)),
        _full((W, HID)),
        _full((1, HID)),
        _full((HID, HID)),
        _full((1, HID)),
    ],
    out_specs=[
        pl.BlockSpec((_NB, HID), lambda i: (i, 0)),
        pl.BlockSpec((_NB, XW), lambda i: (i, 0)),
    ],
    out_shape=[
        jax.ShapeDtypeStruct((N, HID), jnp.float32),
        jax.ShapeDtypeStruct((N, XW), jnp.float32),
    ],
)


def _pad_rowp(rowx, epw):
    spw = -(-epw // CHS)
    return jnp.pad(rowx.reshape(NW, epw), ((0, 0), (0, spw * CHS - epw)),
                   constant_values=N).reshape(NW, spw, CHS)


def kernel(h, x, edges, edge_attr, params):
    row = edges[0]
    col = edges[1]
    rowA, rowB = row[:EA], row[EA:]
    colA, colB = col[:EA], col[EA:]
    eaA, eaB = edge_attr[:EA], edge_attr[EA:]
    rowpA = _pad_rowp(rowA, EPWA)
    rowpB = _pad_rowp(rowB, EPWB)
    xp = jnp.pad(x, ((0, 0), (0, XW - 3)))
    zacc = jnp.zeros((NACC, W), jnp.float32)

    def b2d(b):
        return b.reshape(1, -1)

    def padw(m):
        return jnp.pad(m, ((0, 0), (0, W - m.shape[1])))

    hcur = _embed_in(h, params["emb_in"]["w"], b2d(params["emb_in"]["b"]))
    xcur = xp
    for lp in params["layers"]:
        w0 = lp["edge_mlp0"]["w"]
        w0a, w0b = w0[0:HID], w0[HID:2 * HID]
        w0r = padw(w0[2 * HID:2 * HID + 1])
        w0e = padw(w0[2 * HID + 1:])
        w1e = jnp.pad(lp["edge_mlp1"]["w"],
                      ((0, W - HID), (0, W - HID)))
        b1e = padw(b2d(lp["edge_mlp1"]["b"]))
        wc0e = jnp.pad(lp["coord_mlp0"]["w"], ((0, W - HID), (0, 0)))
        ta, tb = _prep(hcur, xcur, w0a, w0b, b2d(lp["edge_mlp0"]["b"]))
        mlp_args = (w0r, w0e, w1e, b1e,
                    wc0e, b2d(lp["coord_mlp0"]["b"]),
                    lp["coord_mlp1"]["w"].reshape(1, HID))
        gaA, gbA = _sc_gather(EPWA, rowA, colA, ta, tb)
        gaB, gbB = _sc_gather(EPWB, rowB, colB, ta, tb)
        oA = _get_edge(EA)(gaA, gbA, eaA, *mlp_args)
        oB = _get_edge(EB)(gaB, gbB, eaB, *mlp_args)
        pA = _sc_scatter(EPWA, rowpA, oA, zacc)
        pB = _sc_scatter(EPWB, rowpB, oB, zacc)
        nw0 = lp["node_mlp0"]["w"]
        wn0be = jnp.pad(nw0[HID:], ((0, W - HID), (0, 0)))
        hcur, xcur = _node(hcur, pA, pB, xcur,
                           nw0[:HID], wn0be, b2d(lp["node_mlp0"]["b"]),
                           lp["node_mlp1"]["w"], b2d(lp["node_mlp1"]["b"]))
    hout = _embed_out(hcur, params["emb_out"]["w"], b2d(params["emb_out"]["b"]))
    return (hout, xcur[:, :3])
